# baseline (speedup 1.0000x reference)
import functools

import jax
import jax.numpy as jnp
from jax import lax
from jax.experimental import pallas as pl
from jax.experimental.pallas import tpu as pltpu
from jax.experimental.pallas import tpu_sc as plsc

N = 10000
N_PAD = 10240
E = 320000
E_PAD = 327680
EPW = E_PAD // 32
IDX_ROWS = EPW // 128
CHUNK = 512
NSUB = CHUNK // 128
EPW_C0 = E_PAD // 16
NCHUNKS0 = EPW_C0 // CHUNK
IDX_ROWS_MAX = EPW_C0 // 128
DCHUNK = 1024
DNSUB = DCHUNK // 128
DNCHUNKS = EPW // DCHUNK
D_IN = 128
D_HID = 64
NCLS = 40
NCLS_PAD = 48
ROWS_PER_TILE = N_PAD // 16
BLK = 512
GRID = N_PAD // BLK


def _sc_mesh():
    return plsc.VectorSubcoreMesh(core_axis_name="c", subcore_axis_name="s")


_SC_PARAMS = pltpu.CompilerParams(use_tc_tiling_on_sc=False)


def _deg_kernel(packed2d, ones_blk, zeros8):
    @functools.partial(
        pl.kernel,
        out_type=jax.ShapeDtypeStruct((2, N_PAD, 8), jnp.float32),
        mesh=_sc_mesh(),
        scratch_types=[
            pltpu.VMEM((DNSUB, 128), jnp.int32),
            pltpu.VMEM((DNSUB, 128), jnp.int32),
            pltpu.VMEM((128, 8), jnp.float32),
            pltpu.VMEM_SHARED((N_PAD, 8), jnp.float32),
        ],
        compiler_params=_SC_PARAMS,
    )
    def k(pk_hbm, ones_hbm, z_hbm, out_hbm, pk_v, idx_v, ones_v, deg_sh):
        c = lax.axis_index("c")
        s = lax.axis_index("s")
        wid = s * 2 + c
        pltpu.sync_copy(ones_hbm, ones_v)
        pltpu.sync_copy(z_hbm.at[pl.ds(s * ROWS_PER_TILE, ROWS_PER_TILE)],
                        deg_sh.at[pl.ds(s * ROWS_PER_TILE, ROWS_PER_TILE)])
        plsc.subcore_barrier()
        base = wid * IDX_ROWS

        def body(i, _):
            pltpu.sync_copy(pk_hbm.at[pl.ds(base + i * DNSUB, DNSUB)], pk_v)
            for j in range(DNSUB):
                for t in range(8):
                    v = pk_v[j, pl.ds(t * 16, 16)]
                    idx_v[j, pl.ds(t * 16, 16)] = jnp.bitwise_and(v, 0xFFFF)
            for j in range(DNSUB):
                pltpu.sync_copy(ones_v, deg_sh.at[idx_v.at[j]], add=True)
            return _

        lax.fori_loop(0, DNCHUNKS, body, 0)
        plsc.subcore_barrier()
        pltpu.sync_copy(deg_sh.at[pl.ds(s * ROWS_PER_TILE, ROWS_PER_TILE)],
                        out_hbm.at[c, pl.ds(s * ROWS_PER_TILE, ROWS_PER_TILE)])

    return k(packed2d, ones_blk, zeros8)


def _propagate(g_pad, packed2d, zeros_nd, d):
    @functools.partial(
        pl.kernel,
        out_type=jax.ShapeDtypeStruct((N_PAD, d), jnp.float32),
        mesh=_sc_mesh(),
        scratch_types=[
            pltpu.VMEM((IDX_ROWS_MAX, 128), jnp.int32),
            pltpu.VMEM((NSUB, 128), jnp.int32),
            pltpu.VMEM((NSUB, 128), jnp.int32),
            pltpu.VMEM((NSUB, 128), jnp.int32),
            pltpu.VMEM((NSUB, 128), jnp.int32),
            pltpu.VMEM((CHUNK, d), jnp.float32),
            pltpu.VMEM((CHUNK, d), jnp.float32),
            pltpu.VMEM_SHARED((N_PAD, d), jnp.float32),
            pltpu.SemaphoreType.DMA,
            pltpu.SemaphoreType.DMA,
            pltpu.SemaphoreType.DMA,
            pltpu.SemaphoreType.DMA,
        ],
        compiler_params=_SC_PARAMS,
    )
    def k(g_hbm, pk_hbm, z_hbm, out_hbm, pk_v, src0, src1, dst0, dst1,
          rows0, rows1, acc_sh, sem_g0, sem_g1, sem_s0, sem_s1):
        c = lax.axis_index("c")
        s = lax.axis_index("s")

        @pl.when(c == 0)
        def _():
            base = s * IDX_ROWS_MAX
            pltpu.sync_copy(pk_hbm.at[pl.ds(base, IDX_ROWS_MAX)], pk_v)
            pltpu.sync_copy(z_hbm.at[pl.ds(s * ROWS_PER_TILE, ROWS_PER_TILE)],
                            acc_sh.at[pl.ds(s * ROWS_PER_TILE, ROWS_PER_TILE)])
            plsc.subcore_barrier()

            rows = (rows0, rows1)
            src_i = (src0, src1)
            dst_i = (dst0, dst1)
            sem_g = (sem_g0, sem_g1)
            sem_s = (sem_s0, sem_s1)

            def unpack(ci, b):
                for j in range(NSUB):
                    row = ci * NSUB + j
                    for t in range(8):
                        v = pk_v[row, pl.ds(t * 16, 16)]
                        src_i[b][j, pl.ds(t * 16, 16)] = jnp.right_shift(v, 16)
                        dst_i[b][j, pl.ds(t * 16, 16)] = jnp.bitwise_and(
                            v, 0xFFFF)

            def fire_g(b):
                for j in range(NSUB):
                    pltpu.async_copy(g_hbm.at[src_i[b].at[j]],
                                     rows[b].at[pl.ds(j * 128, 128)], sem_g[b])

            def wait_g(b):
                for j in range(NSUB):
                    pltpu.make_async_copy(g_hbm.at[src_i[b].at[j]],
                                          rows[b].at[pl.ds(j * 128, 128)],
                                          sem_g[b]).wait()

            def fire_s(b):
                for j in range(NSUB):
                    pltpu.async_copy(rows[b].at[pl.ds(j * 128, 128)],
                                     acc_sh.at[dst_i[b].at[j]], sem_s[b],
                                     add=True)

            def wait_s(b):
                for j in range(NSUB):
                    pltpu.make_async_copy(rows[b].at[pl.ds(j * 128, 128)],
                                          acc_sh.at[dst_i[b].at[j]],
                                          sem_s[b]).wait()

            unpack(0, 0)
            fire_g(0)
            wait_g(0)
            fire_s(0)
            unpack(1, 1)
            fire_g(1)

            def super_body(k2, _):
                c1 = 2 * k2 + 1
                wait_g(1)
                fire_s(1)
                wait_s(0)
                unpack(c1 + 1, 0)
                fire_g(0)
                wait_g(0)
                fire_s(0)
                wait_s(1)
                unpack(c1 + 2, 1)
                fire_g(1)
                return _

            lax.fori_loop(0, (NCHUNKS0 - 2) // 2, super_body, 0)
            wait_g(1)
            fire_s(1)
            wait_s(0)
            wait_s(1)
            plsc.subcore_barrier()
            pltpu.sync_copy(
                acc_sh.at[pl.ds(s * ROWS_PER_TILE, ROWS_PER_TILE)],
                out_hbm.at[pl.ds(s * ROWS_PER_TILE, ROWS_PER_TILE)])

    return k(g_pad, packed2d, zeros_nd)


def _tc_a(x_pad, w1, deg_p):
    def body(x_ref, w_ref, dp_ref, g_ref, dis_ref):
        deg = dp_ref[0, :, 0:1] + dp_ref[1, :, 0:1] + 1.0
        dis = lax.rsqrt(deg)
        h = jnp.dot(x_ref[...], w_ref[...], preferred_element_type=jnp.float32)
        g_ref[...] = h * dis
        dis_ref[...] = dis

    return pl.pallas_call(
        body,
        grid=(GRID,),
        in_specs=[
            pl.BlockSpec((BLK, D_IN), lambda i: (i, 0)),
            pl.BlockSpec((D_IN, D_HID), lambda i: (0, 0)),
            pl.BlockSpec((2, BLK, 8), lambda i: (0, i, 0)),
        ],
        out_specs=[
            pl.BlockSpec((BLK, D_HID), lambda i: (i, 0)),
            pl.BlockSpec((BLK, 1), lambda i: (i, 0)),
        ],
        out_shape=[
            jax.ShapeDtypeStruct((N_PAD, D_HID), jnp.float32),
            jax.ShapeDtypeStruct((N_PAD, 1), jnp.float32),
        ],
    )(x_pad, w1, deg_p)


def _tc_b(acc1, g1, dis, b1, w2p):
    def body(a_ref, g_ref, dis_ref, b_ref, w_ref, g2_ref):
        srow = a_ref[...] + g_ref[...]
        h1 = jnp.maximum(srow * dis_ref[...] + b_ref[...], 0.0)
        g2 = jnp.dot(h1, w_ref[...], preferred_element_type=jnp.float32)
        g2 = g2 * dis_ref[...]
        rows = pl.program_id(0) * BLK + lax.broadcasted_iota(
            jnp.int32, (BLK, 1), 0)
        g2_ref[...] = jnp.where(rows < N, g2, 0.0)

    return pl.pallas_call(
        body,
        grid=(GRID,),
        in_specs=[
            pl.BlockSpec((BLK, D_HID), lambda i: (i, 0)),
            pl.BlockSpec((BLK, D_HID), lambda i: (i, 0)),
            pl.BlockSpec((BLK, 1), lambda i: (i, 0)),
            pl.BlockSpec((1, D_HID), lambda i: (0, 0)),
            pl.BlockSpec((D_HID, NCLS_PAD), lambda i: (0, 0)),
        ],
        out_specs=pl.BlockSpec((BLK, NCLS_PAD), lambda i: (i, 0)),
        out_shape=jax.ShapeDtypeStruct((N_PAD, NCLS_PAD), jnp.float32),
    )(acc1, g1, dis, b1, w2p)


def _tc_c(acc2, g2, dis, b2p):
    def body(a_ref, g_ref, dis_ref, b_ref, out_ref):
        srow = (a_ref[...] + g_ref[...]) * dis_ref[...] + b_ref[...]
        cols = lax.broadcasted_iota(jnp.int32, (BLK, NCLS_PAD), 1)
        valid = cols < NCLS
        m = jnp.max(jnp.where(valid, srow, -jnp.inf), axis=1, keepdims=True)
        ex = jnp.where(valid, jnp.exp(srow - m), 0.0)
        lse = jnp.log(jnp.sum(ex, axis=1, keepdims=True)) + m
        out_ref[...] = (srow - lse)[:, :NCLS]

    return pl.pallas_call(
        body,
        grid=(GRID,),
        in_specs=[
            pl.BlockSpec((BLK, NCLS_PAD), lambda i: (i, 0)),
            pl.BlockSpec((BLK, NCLS_PAD), lambda i: (i, 0)),
            pl.BlockSpec((BLK, 1), lambda i: (i, 0)),
            pl.BlockSpec((1, NCLS_PAD), lambda i: (0, 0)),
        ],
        out_specs=pl.BlockSpec((BLK, NCLS), lambda i: (i, 0)),
        out_shape=jax.ShapeDtypeStruct((N_PAD, NCLS), jnp.float32),
    )(acc2, g2, dis, b2p)


def kernel(x, edge_index, W1, b1, W2, b2):
    pad_e = E_PAD - E
    src = jnp.concatenate([edge_index[0], jnp.full((pad_e,), N, jnp.int32)])
    dst = jnp.concatenate([edge_index[1], jnp.full((pad_e,), N, jnp.int32)])
    packed = jnp.bitwise_or(jnp.left_shift(src, 16), dst).reshape(-1, 128)
    x_pad = jnp.zeros((N_PAD, D_IN), jnp.float32).at[:N].set(x)
    w2p = jnp.zeros((D_HID, NCLS_PAD), jnp.float32).at[:, :NCLS].set(W2)
    b1r = b1.reshape(1, D_HID)
    b2p = jnp.zeros((1, NCLS_PAD), jnp.float32).at[0, :NCLS].set(b2)
    ones_blk = jnp.ones((128, 8), jnp.float32)
    zeros8 = jnp.zeros((N_PAD, 8), jnp.float32)
    zeros_hid = jnp.zeros((N_PAD, D_HID), jnp.float32)
    zeros_cls = jnp.zeros((N_PAD, NCLS_PAD), jnp.float32)

    deg_p = _deg_kernel(packed, ones_blk, zeros8)
    g1, dis = _tc_a(x_pad, W1, deg_p)
    acc1 = _propagate(g1, packed, zeros_hid, D_HID)
    g2 = _tc_b(acc1, g1, dis, b1r, w2p)
    acc2 = _propagate(g2, packed, zeros_cls, NCLS_PAD)
    out = _tc_c(acc2, g2, dis, b2p)
    return out[:N]


# speedup vs baseline: 1.3090x; 1.3090x over previous
import functools

import jax
import jax.numpy as jnp
from jax import lax
from jax.experimental import pallas as pl
from jax.experimental.pallas import tpu as pltpu
from jax.experimental.pallas import tpu_sc as plsc

N = 10000
N_PAD = 10240
E = 320000
E_PAD = 327680
EPW = E_PAD // 32
IDX_ROWS = EPW // 128
CHUNK = 512
NSUB = CHUNK // 128
EPW_C0 = 18432
EPW_C1 = 2048
IDX_ROWS_MAX = max(EPW_C0, EPW_C1) // 128
DCHUNK = 1024
DNSUB = DCHUNK // 128
DNCHUNKS = EPW // DCHUNK
D_IN = 128
D_HID = 64
NCLS = 40
NCLS_PAD = 48
ROWS_PER_TILE = N_PAD // 16
BLK = 512
GRID = N_PAD // BLK


def _sc_mesh():
    return plsc.VectorSubcoreMesh(core_axis_name="c", subcore_axis_name="s")


_SC_PARAMS = pltpu.CompilerParams(use_tc_tiling_on_sc=False)


def _deg_kernel(packed2d, ones_blk, zeros8):
    @functools.partial(
        pl.kernel,
        out_type=jax.ShapeDtypeStruct((2, N_PAD, 8), jnp.float32),
        mesh=_sc_mesh(),
        scratch_types=[
            pltpu.VMEM((DNSUB, 128), jnp.int32),
            pltpu.VMEM((DNSUB, 128), jnp.int32),
            pltpu.VMEM((128, 8), jnp.float32),
            pltpu.VMEM_SHARED((N_PAD, 8), jnp.float32),
        ],
        compiler_params=_SC_PARAMS,
    )
    def k(pk_hbm, ones_hbm, z_hbm, out_hbm, pk_v, idx_v, ones_v, deg_sh):
        c = lax.axis_index("c")
        s = lax.axis_index("s")
        wid = s * 2 + c
        pltpu.sync_copy(ones_hbm, ones_v)
        pltpu.sync_copy(z_hbm.at[pl.ds(s * ROWS_PER_TILE, ROWS_PER_TILE)],
                        deg_sh.at[pl.ds(s * ROWS_PER_TILE, ROWS_PER_TILE)])
        plsc.subcore_barrier()
        base = wid * IDX_ROWS

        def body(i, _):
            pltpu.sync_copy(pk_hbm.at[pl.ds(base + i * DNSUB, DNSUB)], pk_v)
            for j in range(DNSUB):
                for t in range(8):
                    v = pk_v[j, pl.ds(t * 16, 16)]
                    idx_v[j, pl.ds(t * 16, 16)] = jnp.bitwise_and(v, 0xFFFF)
            for j in range(DNSUB):
                pltpu.sync_copy(ones_v, deg_sh.at[idx_v.at[j]], add=True)
            return _

        lax.fori_loop(0, DNCHUNKS, body, 0)
        plsc.subcore_barrier()
        pltpu.sync_copy(deg_sh.at[pl.ds(s * ROWS_PER_TILE, ROWS_PER_TILE)],
                        out_hbm.at[c, pl.ds(s * ROWS_PER_TILE, ROWS_PER_TILE)])

    return k(packed2d, ones_blk, zeros8)


def _propagate(g_pad, packed2d, zeros_nd, d):
    @functools.partial(
        pl.kernel,
        out_type=jax.ShapeDtypeStruct((2, N_PAD, d), jnp.float32),
        mesh=_sc_mesh(),
        scratch_types=[
            pltpu.VMEM((IDX_ROWS_MAX, 128), jnp.int32),
            pltpu.VMEM((NSUB, 128), jnp.int32),
            pltpu.VMEM((NSUB, 128), jnp.int32),
            pltpu.VMEM((NSUB, 128), jnp.int32),
            pltpu.VMEM((NSUB, 128), jnp.int32),
            pltpu.VMEM((CHUNK, d), jnp.float32),
            pltpu.VMEM((CHUNK, d), jnp.float32),
            pltpu.VMEM_SHARED((N_PAD, d), jnp.float32),
            pltpu.SemaphoreType.DMA,
            pltpu.SemaphoreType.DMA,
            pltpu.SemaphoreType.DMA,
            pltpu.SemaphoreType.DMA,
        ],
        compiler_params=_SC_PARAMS,
    )
    def k(g_hbm, pk_hbm, z_hbm, out_hbm, pk_v, src0, src1, dst0, dst1,
          rows0, rows1, acc_sh, sem_g0, sem_g1, sem_s0, sem_s1):
        c = lax.axis_index("c")
        s = lax.axis_index("s")
        nchunks = jnp.where(c == 0, EPW_C0 // CHUNK, EPW_C1 // CHUNK)
        base = jnp.where(c == 0, s * (EPW_C0 // 128),
                         16 * (EPW_C0 // 128) + s * (EPW_C1 // 128))

        @pl.when(c == 0)
        def _():
            r = EPW_C0 // 128
            pltpu.sync_copy(pk_hbm.at[pl.ds(base, r)], pk_v.at[pl.ds(0, r)])

        @pl.when(c == 1)
        def _():
            r = EPW_C1 // 128
            pltpu.sync_copy(pk_hbm.at[pl.ds(base, r)], pk_v.at[pl.ds(0, r)])

        pltpu.sync_copy(z_hbm.at[pl.ds(s * ROWS_PER_TILE, ROWS_PER_TILE)],
                        acc_sh.at[pl.ds(s * ROWS_PER_TILE, ROWS_PER_TILE)])
        plsc.subcore_barrier()

        rows = (rows0, rows1)
        src_i = (src0, src1)
        dst_i = (dst0, dst1)
        sem_g = (sem_g0, sem_g1)
        sem_s = (sem_s0, sem_s1)

        def unpack(ci, b):
            for j in range(NSUB):
                row = ci * NSUB + j
                for t in range(8):
                    v = pk_v[row, pl.ds(t * 16, 16)]
                    src_i[b][j, pl.ds(t * 16, 16)] = jnp.right_shift(v, 16)
                    dst_i[b][j, pl.ds(t * 16, 16)] = jnp.bitwise_and(v, 0xFFFF)

        def fire_g(b):
            for j in range(NSUB):
                pltpu.async_copy(g_hbm.at[src_i[b].at[j]],
                                 rows[b].at[pl.ds(j * 128, 128)], sem_g[b])

        def wait_g(b):
            for j in range(NSUB):
                pltpu.make_async_copy(g_hbm.at[src_i[b].at[j]],
                                      rows[b].at[pl.ds(j * 128, 128)],
                                      sem_g[b]).wait()

        def fire_s(b):
            for j in range(NSUB):
                pltpu.async_copy(rows[b].at[pl.ds(j * 128, 128)],
                                 acc_sh.at[dst_i[b].at[j]], sem_s[b], add=True)

        def wait_s(b):
            for j in range(NSUB):
                pltpu.make_async_copy(rows[b].at[pl.ds(j * 128, 128)],
                                      acc_sh.at[dst_i[b].at[j]],
                                      sem_s[b]).wait()

        unpack(0, 0)
        fire_g(0)
        wait_g(0)
        fire_s(0)
        unpack(1, 1)
        fire_g(1)

        def super_body(k2, _):
            c1 = 2 * k2 + 1
            wait_g(1)
            fire_s(1)
            wait_s(0)
            unpack(c1 + 1, 0)
            fire_g(0)
            wait_g(0)
            fire_s(0)
            wait_s(1)
            unpack(c1 + 2, 1)
            fire_g(1)
            return _

        lax.fori_loop(0, (nchunks - 2) // 2, super_body, 0)
        wait_g(1)
        fire_s(1)
        wait_s(0)
        wait_s(1)
        plsc.subcore_barrier()
        pltpu.sync_copy(acc_sh.at[pl.ds(s * ROWS_PER_TILE, ROWS_PER_TILE)],
                        out_hbm.at[c, pl.ds(s * ROWS_PER_TILE, ROWS_PER_TILE)])

    return k(g_pad, packed2d, zeros_nd)


def _tc_a(x_pad, w1, deg_p):
    def body(x_ref, w_ref, dp_ref, g_ref, dis_ref):
        deg = dp_ref[0, :, 0:1] + dp_ref[1, :, 0:1] + 1.0
        dis = lax.rsqrt(deg)
        h = jnp.dot(x_ref[...], w_ref[...], preferred_element_type=jnp.float32)
        g_ref[...] = h * dis
        dis_ref[...] = dis

    return pl.pallas_call(
        body,
        grid=(GRID,),
        in_specs=[
            pl.BlockSpec((BLK, D_IN), lambda i: (i, 0)),
            pl.BlockSpec((D_IN, D_HID), lambda i: (0, 0)),
            pl.BlockSpec((2, BLK, 8), lambda i: (0, i, 0)),
        ],
        out_specs=[
            pl.BlockSpec((BLK, D_HID), lambda i: (i, 0)),
            pl.BlockSpec((BLK, 1), lambda i: (i, 0)),
        ],
        out_shape=[
            jax.ShapeDtypeStruct((N_PAD, D_HID), jnp.float32),
            jax.ShapeDtypeStruct((N_PAD, 1), jnp.float32),
        ],
    )(x_pad, w1, deg_p)


def _tc_b(acc1, g1, dis, b1, w2p):
    def body(a_ref, g_ref, dis_ref, b_ref, w_ref, g2_ref):
        srow = a_ref[0] + a_ref[1] + g_ref[...]
        h1 = jnp.maximum(srow * dis_ref[...] + b_ref[...], 0.0)
        g2 = jnp.dot(h1, w_ref[...], preferred_element_type=jnp.float32)
        g2 = g2 * dis_ref[...]
        rows = pl.program_id(0) * BLK + lax.broadcasted_iota(
            jnp.int32, (BLK, 1), 0)
        g2_ref[...] = jnp.where(rows < N, g2, 0.0)

    return pl.pallas_call(
        body,
        grid=(GRID,),
        in_specs=[
            pl.BlockSpec((2, BLK, D_HID), lambda i: (0, i, 0)),
            pl.BlockSpec((BLK, D_HID), lambda i: (i, 0)),
            pl.BlockSpec((BLK, 1), lambda i: (i, 0)),
            pl.BlockSpec((1, D_HID), lambda i: (0, 0)),
            pl.BlockSpec((D_HID, NCLS_PAD), lambda i: (0, 0)),
        ],
        out_specs=pl.BlockSpec((BLK, NCLS_PAD), lambda i: (i, 0)),
        out_shape=jax.ShapeDtypeStruct((N_PAD, NCLS_PAD), jnp.float32),
    )(acc1, g1, dis, b1, w2p)


def _tc_c(acc2, g2, dis, b2p):
    def body(a_ref, g_ref, dis_ref, b_ref, out_ref):
        srow = (a_ref[0] + a_ref[1] + g_ref[...]) * dis_ref[...] + b_ref[...]
        cols = lax.broadcasted_iota(jnp.int32, (BLK, NCLS_PAD), 1)
        valid = cols < NCLS
        m = jnp.max(jnp.where(valid, srow, -jnp.inf), axis=1, keepdims=True)
        ex = jnp.where(valid, jnp.exp(srow - m), 0.0)
        lse = jnp.log(jnp.sum(ex, axis=1, keepdims=True)) + m
        out_ref[...] = (srow - lse)[:, :NCLS]

    return pl.pallas_call(
        body,
        grid=(GRID,),
        in_specs=[
            pl.BlockSpec((2, BLK, NCLS_PAD), lambda i: (0, i, 0)),
            pl.BlockSpec((BLK, NCLS_PAD), lambda i: (i, 0)),
            pl.BlockSpec((BLK, 1), lambda i: (i, 0)),
            pl.BlockSpec((1, NCLS_PAD), lambda i: (0, 0)),
        ],
        out_specs=pl.BlockSpec((BLK, NCLS), lambda i: (i, 0)),
        out_shape=jax.ShapeDtypeStruct((N_PAD, NCLS), jnp.float32),
    )(acc2, g2, dis, b2p)


def kernel(x, edge_index, W1, b1, W2, b2):
    pad_e = E_PAD - E
    src = jnp.concatenate([edge_index[0], jnp.full((pad_e,), N, jnp.int32)])
    dst = jnp.concatenate([edge_index[1], jnp.full((pad_e,), N, jnp.int32)])
    packed = jnp.bitwise_or(jnp.left_shift(src, 16), dst).reshape(-1, 128)
    x_pad = jnp.zeros((N_PAD, D_IN), jnp.float32).at[:N].set(x)
    w2p = jnp.zeros((D_HID, NCLS_PAD), jnp.float32).at[:, :NCLS].set(W2)
    b1r = b1.reshape(1, D_HID)
    b2p = jnp.zeros((1, NCLS_PAD), jnp.float32).at[0, :NCLS].set(b2)
    ones_blk = jnp.ones((128, 8), jnp.float32)
    zeros8 = jnp.zeros((N_PAD, 8), jnp.float32)
    zeros_hid = jnp.zeros((N_PAD, D_HID), jnp.float32)
    zeros_cls = jnp.zeros((N_PAD, NCLS_PAD), jnp.float32)

    deg_p = _deg_kernel(packed, ones_blk, zeros8)
    g1, dis = _tc_a(x_pad, W1, deg_p)
    acc1 = _propagate(g1, packed, zeros_hid, D_HID)
    g2 = _tc_b(acc1, g1, dis, b1r, w2p)
    acc2 = _propagate(g2, packed, zeros_cls, NCLS_PAD)
    out = _tc_c(acc2, g2, dis, b2p)
    return out[:N]


# speedup vs baseline: 1.3155x; 1.0050x over previous
import functools

import jax
import jax.numpy as jnp
from jax import lax
from jax.experimental import pallas as pl
from jax.experimental.pallas import tpu as pltpu
from jax.experimental.pallas import tpu_sc as plsc

N = 10000
N_PAD = 10240
E = 320000
E_PAD = 327680
EPW = E_PAD // 32
IDX_ROWS = EPW // 128
CHUNK = 512
NSUB = CHUNK // 128
EPW_C0 = 19456
EPW_C1 = 1024
IDX_ROWS_MAX = max(EPW_C0, EPW_C1) // 128
DCHUNK = 1024
DNSUB = DCHUNK // 128
DNCHUNKS = EPW // DCHUNK
D_IN = 128
D_HID = 64
NCLS = 40
NCLS_PAD = 48
ROWS_PER_TILE = N_PAD // 16
BLK = 512
GRID = N_PAD // BLK


def _sc_mesh():
    return plsc.VectorSubcoreMesh(core_axis_name="c", subcore_axis_name="s")


_SC_PARAMS = pltpu.CompilerParams(use_tc_tiling_on_sc=False)


def _deg_kernel(packed2d, ones_blk, zeros8):
    @functools.partial(
        pl.kernel,
        out_type=jax.ShapeDtypeStruct((2, N_PAD, 8), jnp.float32),
        mesh=_sc_mesh(),
        scratch_types=[
            pltpu.VMEM((DNSUB, 128), jnp.int32),
            pltpu.VMEM((DNSUB, 128), jnp.int32),
            pltpu.VMEM((128, 8), jnp.float32),
            pltpu.VMEM_SHARED((N_PAD, 8), jnp.float32),
        ],
        compiler_params=_SC_PARAMS,
    )
    def k(pk_hbm, ones_hbm, z_hbm, out_hbm, pk_v, idx_v, ones_v, deg_sh):
        c = lax.axis_index("c")
        s = lax.axis_index("s")
        wid = s * 2 + c
        pltpu.sync_copy(ones_hbm, ones_v)
        pltpu.sync_copy(z_hbm.at[pl.ds(s * ROWS_PER_TILE, ROWS_PER_TILE)],
                        deg_sh.at[pl.ds(s * ROWS_PER_TILE, ROWS_PER_TILE)])
        plsc.subcore_barrier()
        base = wid * IDX_ROWS

        def body(i, _):
            pltpu.sync_copy(pk_hbm.at[pl.ds(base + i * DNSUB, DNSUB)], pk_v)
            for j in range(DNSUB):
                for t in range(8):
                    v = pk_v[j, pl.ds(t * 16, 16)]
                    idx_v[j, pl.ds(t * 16, 16)] = jnp.bitwise_and(v, 0xFFFF)
            for j in range(DNSUB):
                pltpu.sync_copy(ones_v, deg_sh.at[idx_v.at[j]], add=True)
            return _

        lax.fori_loop(0, DNCHUNKS, body, 0)
        plsc.subcore_barrier()
        pltpu.sync_copy(deg_sh.at[pl.ds(s * ROWS_PER_TILE, ROWS_PER_TILE)],
                        out_hbm.at[c, pl.ds(s * ROWS_PER_TILE, ROWS_PER_TILE)])

    return k(packed2d, ones_blk, zeros8)


def _propagate(g_pad, packed2d, zeros_nd, d):
    @functools.partial(
        pl.kernel,
        out_type=jax.ShapeDtypeStruct((2, N_PAD, d), jnp.float32),
        mesh=_sc_mesh(),
        scratch_types=[
            pltpu.VMEM((IDX_ROWS_MAX, 128), jnp.int32),
            pltpu.VMEM((NSUB, 128), jnp.int32),
            pltpu.VMEM((NSUB, 128), jnp.int32),
            pltpu.VMEM((NSUB, 128), jnp.int32),
            pltpu.VMEM((NSUB, 128), jnp.int32),
            pltpu.VMEM((CHUNK, d), jnp.float32),
            pltpu.VMEM((CHUNK, d), jnp.float32),
            pltpu.VMEM_SHARED((N_PAD, d), jnp.float32),
            pltpu.SemaphoreType.DMA,
            pltpu.SemaphoreType.DMA,
            pltpu.SemaphoreType.DMA,
            pltpu.SemaphoreType.DMA,
        ],
        compiler_params=_SC_PARAMS,
    )
    def k(g_hbm, pk_hbm, z_hbm, out_hbm, pk_v, src0, src1, dst0, dst1,
          rows0, rows1, acc_sh, sem_g0, sem_g1, sem_s0, sem_s1):
        c = lax.axis_index("c")
        s = lax.axis_index("s")
        nchunks = jnp.where(c == 0, EPW_C0 // CHUNK, EPW_C1 // CHUNK)
        base = jnp.where(c == 0, s * (EPW_C0 // 128),
                         16 * (EPW_C0 // 128) + s * (EPW_C1 // 128))

        @pl.when(c == 0)
        def _():
            r = EPW_C0 // 128
            pltpu.sync_copy(pk_hbm.at[pl.ds(base, r)], pk_v.at[pl.ds(0, r)])

        @pl.when(c == 1)
        def _():
            r = EPW_C1 // 128
            pltpu.sync_copy(pk_hbm.at[pl.ds(base, r)], pk_v.at[pl.ds(0, r)])

        pltpu.sync_copy(z_hbm.at[pl.ds(s * ROWS_PER_TILE, ROWS_PER_TILE)],
                        acc_sh.at[pl.ds(s * ROWS_PER_TILE, ROWS_PER_TILE)])
        plsc.subcore_barrier()

        rows = (rows0, rows1)
        src_i = (src0, src1)
        dst_i = (dst0, dst1)
        sem_g = (sem_g0, sem_g1)
        sem_s = (sem_s0, sem_s1)

        def unpack(ci, b):
            for j in range(NSUB):
                row = ci * NSUB + j
                for t in range(8):
                    v = pk_v[row, pl.ds(t * 16, 16)]
                    src_i[b][j, pl.ds(t * 16, 16)] = jnp.right_shift(v, 16)
                    dst_i[b][j, pl.ds(t * 16, 16)] = jnp.bitwise_and(v, 0xFFFF)

        def fire_g(b):
            for j in range(NSUB):
                pltpu.async_copy(g_hbm.at[src_i[b].at[j]],
                                 rows[b].at[pl.ds(j * 128, 128)], sem_g[b])

        def wait_g(b):
            for j in range(NSUB):
                pltpu.make_async_copy(g_hbm.at[src_i[b].at[j]],
                                      rows[b].at[pl.ds(j * 128, 128)],
                                      sem_g[b]).wait()

        def fire_s(b):
            for j in range(NSUB):
                pltpu.async_copy(rows[b].at[pl.ds(j * 128, 128)],
                                 acc_sh.at[dst_i[b].at[j]], sem_s[b], add=True)

        def wait_s(b):
            for j in range(NSUB):
                pltpu.make_async_copy(rows[b].at[pl.ds(j * 128, 128)],
                                      acc_sh.at[dst_i[b].at[j]],
                                      sem_s[b]).wait()

        unpack(0, 0)
        fire_g(0)
        wait_g(0)
        fire_s(0)
        unpack(1, 1)
        fire_g(1)

        def super_body(k2, _):
            c1 = 2 * k2 + 1
            wait_g(1)
            fire_s(1)
            wait_s(0)
            unpack(c1 + 1, 0)
            fire_g(0)
            wait_g(0)
            fire_s(0)
            wait_s(1)
            unpack(c1 + 2, 1)
            fire_g(1)
            return _

        lax.fori_loop(0, (nchunks - 2) // 2, super_body, 0)
        wait_g(1)
        fire_s(1)
        wait_s(0)
        wait_s(1)
        plsc.subcore_barrier()
        pltpu.sync_copy(acc_sh.at[pl.ds(s * ROWS_PER_TILE, ROWS_PER_TILE)],
                        out_hbm.at[c, pl.ds(s * ROWS_PER_TILE, ROWS_PER_TILE)])

    return k(g_pad, packed2d, zeros_nd)


def _tc_a(x_pad, w1, deg_p):
    def body(x_ref, w_ref, dp_ref, g_ref, dis_ref):
        deg = dp_ref[0, :, 0:1] + dp_ref[1, :, 0:1] + 1.0
        dis = lax.rsqrt(deg)
        h = jnp.dot(x_ref[...], w_ref[...], preferred_element_type=jnp.float32)
        g_ref[...] = h * dis
        dis_ref[...] = dis

    return pl.pallas_call(
        body,
        grid=(GRID,),
        in_specs=[
            pl.BlockSpec((BLK, D_IN), lambda i: (i, 0)),
            pl.BlockSpec((D_IN, D_HID), lambda i: (0, 0)),
            pl.BlockSpec((2, BLK, 8), lambda i: (0, i, 0)),
        ],
        out_specs=[
            pl.BlockSpec((BLK, D_HID), lambda i: (i, 0)),
            pl.BlockSpec((BLK, 1), lambda i: (i, 0)),
        ],
        out_shape=[
            jax.ShapeDtypeStruct((N_PAD, D_HID), jnp.float32),
            jax.ShapeDtypeStruct((N_PAD, 1), jnp.float32),
        ],
    )(x_pad, w1, deg_p)


def _tc_b(acc1, g1, dis, b1, w2p):
    def body(a_ref, g_ref, dis_ref, b_ref, w_ref, g2_ref):
        srow = a_ref[0] + a_ref[1] + g_ref[...]
        h1 = jnp.maximum(srow * dis_ref[...] + b_ref[...], 0.0)
        g2 = jnp.dot(h1, w_ref[...], preferred_element_type=jnp.float32)
        g2 = g2 * dis_ref[...]
        rows = pl.program_id(0) * BLK + lax.broadcasted_iota(
            jnp.int32, (BLK, 1), 0)
        g2_ref[...] = jnp.where(rows < N, g2, 0.0)

    return pl.pallas_call(
        body,
        grid=(GRID,),
        in_specs=[
            pl.BlockSpec((2, BLK, D_HID), lambda i: (0, i, 0)),
            pl.BlockSpec((BLK, D_HID), lambda i: (i, 0)),
            pl.BlockSpec((BLK, 1), lambda i: (i, 0)),
            pl.BlockSpec((1, D_HID), lambda i: (0, 0)),
            pl.BlockSpec((D_HID, NCLS_PAD), lambda i: (0, 0)),
        ],
        out_specs=pl.BlockSpec((BLK, NCLS_PAD), lambda i: (i, 0)),
        out_shape=jax.ShapeDtypeStruct((N_PAD, NCLS_PAD), jnp.float32),
    )(acc1, g1, dis, b1, w2p)


def _tc_c(acc2, g2, dis, b2p):
    def body(a_ref, g_ref, dis_ref, b_ref, out_ref):
        srow = (a_ref[0] + a_ref[1] + g_ref[...]) * dis_ref[...] + b_ref[...]
        cols = lax.broadcasted_iota(jnp.int32, (BLK, NCLS_PAD), 1)
        valid = cols < NCLS
        m = jnp.max(jnp.where(valid, srow, -jnp.inf), axis=1, keepdims=True)
        ex = jnp.where(valid, jnp.exp(srow - m), 0.0)
        lse = jnp.log(jnp.sum(ex, axis=1, keepdims=True)) + m
        out_ref[...] = (srow - lse)[:, :NCLS]

    return pl.pallas_call(
        body,
        grid=(GRID,),
        in_specs=[
            pl.BlockSpec((2, BLK, NCLS_PAD), lambda i: (0, i, 0)),
            pl.BlockSpec((BLK, NCLS_PAD), lambda i: (i, 0)),
            pl.BlockSpec((BLK, 1), lambda i: (i, 0)),
            pl.BlockSpec((1, NCLS_PAD), lambda i: (0, 0)),
        ],
        out_specs=pl.BlockSpec((BLK, NCLS), lambda i: (i, 0)),
        out_shape=jax.ShapeDtypeStruct((N_PAD, NCLS), jnp.float32),
    )(acc2, g2, dis, b2p)


def kernel(x, edge_index, W1, b1, W2, b2):
    pad_e = E_PAD - E
    src = jnp.concatenate([edge_index[0], jnp.full((pad_e,), N, jnp.int32)])
    dst = jnp.concatenate([edge_index[1], jnp.full((pad_e,), N, jnp.int32)])
    packed = jnp.bitwise_or(jnp.left_shift(src, 16), dst).reshape(-1, 128)
    x_pad = jnp.zeros((N_PAD, D_IN), jnp.float32).at[:N].set(x)
    w2p = jnp.zeros((D_HID, NCLS_PAD), jnp.float32).at[:, :NCLS].set(W2)
    b1r = b1.reshape(1, D_HID)
    b2p = jnp.zeros((1, NCLS_PAD), jnp.float32).at[0, :NCLS].set(b2)
    ones_blk = jnp.ones((128, 8), jnp.float32)
    zeros8 = jnp.zeros((N_PAD, 8), jnp.float32)
    zeros_hid = jnp.zeros((N_PAD, D_HID), jnp.float32)
    zeros_cls = jnp.zeros((N_PAD, NCLS_PAD), jnp.float32)

    deg_p = _deg_kernel(packed, ones_blk, zeros8)
    g1, dis = _tc_a(x_pad, W1, deg_p)
    acc1 = _propagate(g1, packed, zeros_hid, D_HID)
    g2 = _tc_b(acc1, g1, dis, b1r, w2p)
    acc2 = _propagate(g2, packed, zeros_cls, NCLS_PAD)
    out = _tc_c(acc2, g2, dis, b2p)
    return out[:N]
